# Initial kernel scaffold; baseline (speedup 1.0000x reference)
import functools

import jax
import jax.numpy as jnp
from jax import lax
from jax.experimental import pallas as pl
from jax.experimental.pallas import tpu as pltpu
from jax.experimental.pallas import tpu_sc as plsc

N_NODES = 10000
N_EDGES = 160000
D_NODE = 128

NC = 2
NS = 16
NW = NC * NS


EPW = N_EDGES // NW
G_CH = 128
G_FULL = EPW // G_CH
G_TAIL = EPW - G_FULL * G_CH

_gather_mesh = plsc.VectorSubcoreMesh(
    core_axis_name="c", subcore_axis_name="s", num_cores=NC, num_subcores=NS)


@functools.partial(
    pl.kernel,
    out_type=jax.ShapeDtypeStruct((N_EDGES, D_NODE), jnp.float32),
    mesh=_gather_mesh,
    scratch_types=[
        pltpu.VMEM((G_CH,), jnp.int32),
        pltpu.VMEM((G_CH, D_NODE), jnp.float32),
        pltpu.VMEM((G_TAIL,), jnp.int32),
        pltpu.VMEM((G_TAIL, D_NODE), jnp.float32),
        pltpu.SemaphoreType.DMA,
    ],
)
def _sc_gather(nodes_hbm, src_hbm, out_hbm, idx_v, rows_v, idx_t, rows_t, sem):
    c = lax.axis_index("c")
    s = lax.axis_index("s")
    wid = s * NC + c
    base = wid * EPW

    def chunk(j, carry):
        off = base + j * G_CH
        pltpu.sync_copy(src_hbm.at[pl.ds(off, G_CH)], idx_v)
        pltpu.async_copy(nodes_hbm.at[idx_v], rows_v, sem).wait()
        pltpu.sync_copy(rows_v, out_hbm.at[pl.ds(off, G_CH), :])
        return carry

    lax.fori_loop(0, G_FULL, chunk, 0)
    off = base + G_FULL * G_CH
    pltpu.sync_copy(src_hbm.at[pl.ds(off, G_TAIL)], idx_t)
    pltpu.async_copy(nodes_hbm.at[idx_t], rows_t, sem).wait()
    pltpu.sync_copy(rows_t, out_hbm.at[pl.ds(off, G_TAIL), :])


BE = 2000


def _tc_body(fw_ref, fe_ref, x_ref, W1_ref, b1_ref, W2_ref, b2_ref,
             W3_ref, b3_ref, m0_ref, m1_ref, m2_ref, m3_ref):
    fw = fw_ref[...]
    h = jax.nn.silu(jnp.dot(fw, W1_ref[...], preferred_element_type=jnp.float32)
                    + b1_ref[...])
    h = jax.nn.silu(jnp.dot(h, W2_ref[...], preferred_element_type=jnp.float32)
                    + b2_ref[...])
    w = jnp.dot(h, W3_ref[...], preferred_element_type=jnp.float32) + b3_ref[...]
    x = x_ref[...]
    u0 = w[:, :D_NODE] * x
    u1 = w[:, D_NODE:] * x
    fe = fe_ref[...]
    m0_ref[...] = u0 * fe[:, 0:1]
    m1_ref[...] = u1 * fe[:, 1:2]
    m2_ref[...] = u1 * fe[:, 2:3]
    m3_ref[...] = u1 * fe[:, 3:4]


def _tc_messages(fw, fe, x_src, W1, b1, W2, b2, W3, b3):
    n_blocks = N_EDGES // BE
    full = lambda shape: pl.BlockSpec(shape, lambda i: (0, 0))
    blk = lambda cols: pl.BlockSpec((BE, cols), lambda i: (i, 0))
    out = pl.pallas_call(
        _tc_body,
        grid=(n_blocks,),
        in_specs=[
            blk(16), blk(4), blk(D_NODE),
            full((16, 64)), full((1, 64)),
            full((64, 64)), full((1, 64)),
            full((64, 256)), full((1, 256)),
        ],
        out_specs=[blk(D_NODE)] * 4,
        out_shape=[jax.ShapeDtypeStruct((N_EDGES, D_NODE), jnp.float32)] * 4,
    )(fw, fe, x_src, W1, b1.reshape(1, 64), W2, b2.reshape(1, 64),
      W3, b3.reshape(1, 256))
    return out


EPT = N_EDGES // NS
S_CH = 80
S_NCH = EPT // S_CH
RPT = N_NODES // NS
ZB = 125

_scatter_mesh = plsc.VectorSubcoreMesh(
    core_axis_name="c", subcore_axis_name="s", num_cores=NC, num_subcores=NS)


@functools.partial(
    pl.kernel,
    out_type=jax.ShapeDtypeStruct((N_NODES, 4, D_NODE), jnp.float32),
    mesh=_scatter_mesh,
    scratch_types=[
        pltpu.VMEM((S_CH,), jnp.int32),
        pltpu.VMEM((S_CH, D_NODE), jnp.float32),
        pltpu.VMEM((ZB, D_NODE), jnp.float32),
        pltpu.VMEM_SHARED((N_NODES, D_NODE), jnp.float32),
    ],
)
def _sc_scatter(m0, m1, m2, m3, tgt_hbm, out_hbm, idx_v, msg_v, zbuf, slab):
    c = lax.axis_index("c")
    tid = lax.axis_index("s")

    def zrow(i, carry):
        r = i // 8
        col = (i % 8) * 16
        zbuf[r, pl.ds(col, 16)] = jnp.zeros((16,), jnp.float32)
        return carry

    lax.fori_loop(0, ZB * 8, zrow, 0)

    planes = (m0, m1, m2, m3)
    for c_val in (0, 1):
        @pl.when(c == c_val)
        def _core():
            for kk in (0, 1):
                k = 2 * c_val + kk
                msrc = planes[k]
                for z in range(RPT // ZB):
                    pltpu.sync_copy(
                        zbuf, slab.at[pl.ds(tid * RPT + z * ZB, ZB), :])
                plsc.subcore_barrier()

                def chunk(j, carry):
                    off = tid * EPT + j * S_CH
                    pltpu.sync_copy(tgt_hbm.at[pl.ds(off, S_CH)], idx_v)
                    pltpu.sync_copy(msrc.at[pl.ds(off, S_CH), :], msg_v)
                    pltpu.sync_copy(msg_v, slab.at[idx_v], add=True)
                    return carry

                lax.fori_loop(0, S_NCH, chunk, 0)
                plsc.subcore_barrier()
                pltpu.sync_copy(
                    slab.at[pl.ds(tid * RPT, RPT), :],
                    out_hbm.at[pl.ds(tid * RPT, RPT), k, :])
                plsc.subcore_barrier()


def kernel(features_node, features_edge, features_weights, edge_index,
           W1, b1, W2, b2, W3, b3):
    src = edge_index[0]
    tgt = edge_index[1]
    x_src = _sc_gather(features_node, src)
    m0, m1, m2, m3 = _tc_messages(features_weights, features_edge, x_src,
                                  W1, b1, W2, b2, W3, b3)
    out = _sc_scatter(m0, m1, m2, m3, tgt)
    return out.reshape(N_NODES, 4 * D_NODE)


# speedup vs baseline: 3.3127x; 3.3127x over previous
import functools

import jax
import jax.numpy as jnp
from jax import lax
from jax.experimental import pallas as pl
from jax.experimental.pallas import tpu as pltpu
from jax.experimental.pallas import tpu_sc as plsc

N_NODES = 10000
N_EDGES = 160000
D_NODE = 128

NC = 2
NS = 16
NW = NC * NS


EPW = N_EDGES // NW
G_CH = 128
G_FULL = EPW // G_CH
G_TAIL = EPW - G_FULL * G_CH

_gather_mesh = plsc.VectorSubcoreMesh(
    core_axis_name="c", subcore_axis_name="s", num_cores=NC, num_subcores=NS)


@functools.partial(
    pl.kernel,
    out_type=jax.ShapeDtypeStruct((N_EDGES, D_NODE), jnp.float32),
    mesh=_gather_mesh,
    scratch_types=[
        pltpu.VMEM((G_CH,), jnp.int32),
        pltpu.VMEM((G_CH, D_NODE), jnp.float32),
        pltpu.VMEM((G_TAIL,), jnp.int32),
        pltpu.VMEM((G_TAIL, D_NODE), jnp.float32),
        pltpu.SemaphoreType.DMA,
    ],
)
def _sc_gather(nodes_hbm, src_hbm, out_hbm, idx_v, rows_v, idx_t, rows_t, sem):
    c = lax.axis_index("c")
    s = lax.axis_index("s")
    wid = s * NC + c
    base = wid * EPW

    def chunk(j, carry):
        off = base + j * G_CH
        pltpu.sync_copy(src_hbm.at[pl.ds(off, G_CH)], idx_v)
        pltpu.async_copy(nodes_hbm.at[idx_v], rows_v, sem).wait()
        pltpu.sync_copy(rows_v, out_hbm.at[pl.ds(off, G_CH), :])
        return carry

    lax.fori_loop(0, G_FULL, chunk, 0)
    off = base + G_FULL * G_CH
    pltpu.sync_copy(src_hbm.at[pl.ds(off, G_TAIL)], idx_t)
    pltpu.async_copy(nodes_hbm.at[idx_t], rows_t, sem).wait()
    pltpu.sync_copy(rows_t, out_hbm.at[pl.ds(off, G_TAIL), :])


BE = 2000


def _tc_body(fw_ref, fe_ref, x_ref, W1_ref, b1_ref, W2_ref, b2_ref,
             W3_ref, b3_ref, m0_ref, m1_ref, m2_ref, m3_ref):
    fw = fw_ref[...]
    h = jax.nn.silu(jnp.dot(fw, W1_ref[...], preferred_element_type=jnp.float32)
                    + b1_ref[...])
    h = jax.nn.silu(jnp.dot(h, W2_ref[...], preferred_element_type=jnp.float32)
                    + b2_ref[...])
    w = jnp.dot(h, W3_ref[...], preferred_element_type=jnp.float32) + b3_ref[...]
    x = x_ref[...]
    u0 = w[:, :D_NODE] * x
    u1 = w[:, D_NODE:] * x
    fe = fe_ref[...]
    m0_ref[...] = u0 * fe[:, 0:1]
    m1_ref[...] = u1 * fe[:, 1:2]
    m2_ref[...] = u1 * fe[:, 2:3]
    m3_ref[...] = u1 * fe[:, 3:4]


def _tc_messages(fw, fe, x_src, W1, b1, W2, b2, W3, b3):
    n_blocks = N_EDGES // BE
    full = lambda shape: pl.BlockSpec(shape, lambda i: (0, 0))
    blk = lambda cols: pl.BlockSpec((BE, cols), lambda i: (i, 0))
    out = pl.pallas_call(
        _tc_body,
        grid=(n_blocks,),
        in_specs=[
            blk(16), blk(4), blk(D_NODE),
            full((16, 64)), full((1, 64)),
            full((64, 64)), full((1, 64)),
            full((64, 256)), full((1, 256)),
        ],
        out_specs=[blk(D_NODE)] * 4,
        out_shape=[jax.ShapeDtypeStruct((N_EDGES, D_NODE), jnp.float32)] * 4,
    )(fw, fe, x_src, W1, b1.reshape(1, 64), W2, b2.reshape(1, 64),
      W3, b3.reshape(1, 256))
    return out


EPT = N_EDGES // NS
S_CH = 80
S_NCH = EPT // S_CH
RPT = N_NODES // NS
ZB = 125

_scatter_mesh = plsc.VectorSubcoreMesh(
    core_axis_name="c", subcore_axis_name="s", num_cores=NC, num_subcores=NS)


@functools.partial(
    pl.kernel,
    out_type=jax.ShapeDtypeStruct((N_NODES, 4, D_NODE), jnp.float32),
    mesh=_scatter_mesh,
    scratch_types=[
        pltpu.VMEM((S_CH,), jnp.int32),
        pltpu.VMEM((S_CH, D_NODE), jnp.float32),
        pltpu.VMEM((ZB, D_NODE), jnp.float32),
        pltpu.VMEM_SHARED((N_NODES, D_NODE), jnp.float32),
    ],
)
def _sc_scatter(m0, m1, m2, m3, tgt_hbm, out_hbm, idx_v, msg_v, zbuf, slab):
    c = lax.axis_index("c")
    tid = lax.axis_index("s")

    def zrow(i, carry):
        r = i // 8
        col = (i % 8) * 16
        zbuf[r, pl.ds(col, 16)] = jnp.zeros((16,), jnp.float32)
        return carry

    lax.fori_loop(0, ZB * 8, zrow, 0)

    planes = (m0, m1, m2, m3)
    for c_val in (0, 1):
        @pl.when(c == c_val)
        def _core():
            for kk in (0, 1):
                k = 2 * c_val + kk
                msrc = planes[k]
                for z in range(RPT // ZB):
                    pltpu.sync_copy(
                        zbuf, slab.at[pl.ds(tid * RPT + z * ZB, ZB), :])
                plsc.subcore_barrier()

                def chunk(j, carry):
                    off = tid * EPT + j * S_CH
                    pltpu.sync_copy(tgt_hbm.at[pl.ds(off, S_CH)], idx_v)
                    pltpu.sync_copy(msrc.at[pl.ds(off, S_CH), :], msg_v)
                    pltpu.sync_copy(msg_v, slab.at[idx_v], add=True)
                    return carry

                lax.fori_loop(0, S_NCH, chunk, 0)
                plsc.subcore_barrier()
                pltpu.sync_copy(
                    slab.at[pl.ds(tid * RPT, RPT), :],
                    out_hbm.at[pl.ds(tid * RPT, RPT), k, :])
                plsc.subcore_barrier()


def kernel(features_node, features_edge, features_weights, edge_index,
           W1, b1, W2, b2, W3, b3):
    src = edge_index[0]
    tgt = edge_index[1]
    x_src = _sc_gather(features_node, src)
    m0, m1, m2, m3 = _tc_messages(features_weights, features_edge, x_src,
                                  W1, b1, W2, b2, W3, b3)
    out = _sc_scatter(m0, m1, m2, m3, tgt)
    out0 = out[:, 0, :]
    out13 = jnp.transpose(out[:, 1:4, :], (0, 2, 1)).reshape(N_NODES, 3 * D_NODE)
    return jnp.concatenate([out0, out13], axis=1)


# speedup vs baseline: 4.7125x; 1.4226x over previous
import functools

import jax
import jax.numpy as jnp
from jax import lax
from jax.experimental import pallas as pl
from jax.experimental.pallas import tpu as pltpu
from jax.experimental.pallas import tpu_sc as plsc

N_NODES = 10000
N_EDGES = 160000
D_NODE = 128

NC = 2
NS = 16
NW = NC * NS


EPW = N_EDGES // NW
G_CH = 128
G_FULL = EPW // G_CH
G_TAIL = EPW - G_FULL * G_CH

_gather_mesh = plsc.VectorSubcoreMesh(
    core_axis_name="c", subcore_axis_name="s", num_cores=NC, num_subcores=NS)


@functools.partial(
    pl.kernel,
    out_type=jax.ShapeDtypeStruct((N_EDGES, D_NODE), jnp.float32),
    mesh=_gather_mesh,
    scratch_types=[
        pltpu.VMEM((EPW,), jnp.int32),
        pltpu.VMEM((G_CH, D_NODE), jnp.float32),
        pltpu.VMEM((G_CH, D_NODE), jnp.float32),
        pltpu.VMEM((G_TAIL, D_NODE), jnp.float32),
        pltpu.SemaphoreType.DMA,
        pltpu.SemaphoreType.DMA,
        pltpu.SemaphoreType.DMA,
    ],
)
def _sc_gather(nodes_hbm, src_hbm, out_hbm, idx_all, b0, b1, bt, s0, s1, st):
    c = lax.axis_index("c")
    s = lax.axis_index("s")
    wid = s * NC + c
    base = wid * EPW

    pltpu.sync_copy(src_hbm.at[pl.ds(base, EPW)], idx_all)
    bufs = (b0, b1)
    sems = (s0, s1)
    pltpu.async_copy(nodes_hbm.at[idx_all.at[pl.ds(0, G_CH)]], b0, s0)
    pltpu.async_copy(nodes_hbm.at[idx_all.at[pl.ds(G_CH, G_CH)]], b1, s1)
    pltpu.async_copy(
        nodes_hbm.at[idx_all.at[pl.ds(G_FULL * G_CH, G_TAIL)]], bt, st)

    def pair(t, carry):
        for b in range(2):
            j = 2 * t + b
            buf, sem = bufs[b], sems[b]

            @pl.when(j < G_FULL)
            def _chunk():
                pltpu.make_async_copy(
                    nodes_hbm.at[idx_all.at[pl.ds(0, G_CH)]], buf, sem).wait()
                pltpu.sync_copy(buf, out_hbm.at[pl.ds(base + j * G_CH, G_CH), :])
                nxt = j + 2

                @pl.when(nxt < G_FULL)
                def _pf():
                    pltpu.async_copy(
                        nodes_hbm.at[idx_all.at[pl.ds(nxt * G_CH, G_CH)]],
                        buf, sem)
        return carry

    lax.fori_loop(0, (G_FULL + 1) // 2, pair, 0)
    pltpu.make_async_copy(
        nodes_hbm.at[idx_all.at[pl.ds(0, G_TAIL)]], bt, st).wait()
    pltpu.sync_copy(bt, out_hbm.at[pl.ds(base + G_FULL * G_CH, G_TAIL), :])


BE = 2000


def _tc_body(fw_ref, fe_ref, x_ref, W1_ref, b1_ref, W2_ref, b2_ref,
             W3_ref, b3_ref, m0_ref, m1_ref, m2_ref, m3_ref):
    fw = fw_ref[...]
    h = jax.nn.silu(jnp.dot(fw, W1_ref[...], preferred_element_type=jnp.float32)
                    + b1_ref[...])
    h = jax.nn.silu(jnp.dot(h, W2_ref[...], preferred_element_type=jnp.float32)
                    + b2_ref[...])
    w = jnp.dot(h, W3_ref[...], preferred_element_type=jnp.float32) + b3_ref[...]
    x = x_ref[...]
    u0 = w[:, :D_NODE] * x
    u1 = w[:, D_NODE:] * x
    fe = fe_ref[...]
    m0_ref[...] = u0 * fe[:, 0:1]
    m1_ref[...] = u1 * fe[:, 1:2]
    m2_ref[...] = u1 * fe[:, 2:3]
    m3_ref[...] = u1 * fe[:, 3:4]


def _tc_messages(fw, fe, x_src, W1, b1, W2, b2, W3, b3):
    n_blocks = N_EDGES // BE
    full = lambda shape: pl.BlockSpec(shape, lambda i: (0, 0))
    blk = lambda cols: pl.BlockSpec((BE, cols), lambda i: (i, 0))
    out = pl.pallas_call(
        _tc_body,
        grid=(n_blocks,),
        in_specs=[
            blk(16), blk(4), blk(D_NODE),
            full((16, 64)), full((1, 64)),
            full((64, 64)), full((1, 64)),
            full((64, 256)), full((1, 256)),
        ],
        out_specs=[blk(D_NODE)] * 4,
        out_shape=[jax.ShapeDtypeStruct((N_EDGES, D_NODE), jnp.float32)] * 4,
    )(fw, fe, x_src, W1, b1.reshape(1, 64), W2, b2.reshape(1, 64),
      W3, b3.reshape(1, 256))
    return out


EPT = N_EDGES // NS
S_CH = 80
S_NCH = EPT // S_CH
RPT = N_NODES // NS
ZB = 25

_scatter_mesh = plsc.VectorSubcoreMesh(
    core_axis_name="c", subcore_axis_name="s", num_cores=NC, num_subcores=NS)


@functools.partial(
    pl.kernel,
    out_type=jax.ShapeDtypeStruct((N_NODES, 4, D_NODE), jnp.float32),
    mesh=_scatter_mesh,
    scratch_types=[
        pltpu.VMEM((S_NCH, S_CH), jnp.int32),
        pltpu.VMEM((S_CH, D_NODE), jnp.float32),
        pltpu.VMEM((S_CH, D_NODE), jnp.float32),
        pltpu.VMEM((ZB, D_NODE), jnp.float32),
        pltpu.VMEM_SHARED((N_NODES, D_NODE), jnp.float32),
        pltpu.SemaphoreType.DMA,
        pltpu.SemaphoreType.DMA,
    ],
)
def _sc_scatter(m0, m1, m2, m3, tgt3_hbm, out_hbm,
                idx_all, mb0, mb1, zbuf, slab, sm0, sm1):
    c = lax.axis_index("c")
    tid = lax.axis_index("s")

    def zrow(i, carry):
        r = i // 8
        col = (i % 8) * 16
        zbuf[r, pl.ds(col, 16)] = jnp.zeros((16,), jnp.float32)
        return carry

    lax.fori_loop(0, ZB * 8, zrow, 0)

    pltpu.sync_copy(tgt3_hbm.at[tid], idx_all)

    bufs = (mb0, mb1)
    sems = (sm0, sm1)
    planes = (m0, m1, m2, m3)
    for c_val in (0, 1):
        @pl.when(c == c_val)
        def _core():
            for kk in (0, 1):
                k = 2 * c_val + kk
                msrc = planes[k]
                ebase = tid * EPT
                for z in range(RPT // ZB):
                    pltpu.sync_copy(
                        zbuf, slab.at[pl.ds(tid * RPT + z * ZB, ZB), :])
                plsc.subcore_barrier()

                pltpu.async_copy(msrc.at[pl.ds(ebase, S_CH), :], mb0, sm0)
                pltpu.async_copy(
                    msrc.at[pl.ds(ebase + S_CH, S_CH), :], mb1, sm1)

                def pair(t, carry):
                    for b in range(2):
                        j = 2 * t + b
                        buf, sem = bufs[b], sems[b]

                        @pl.when(j < S_NCH)
                        def _chunk():
                            pltpu.make_async_copy(
                                msrc.at[pl.ds(0, S_CH), :], buf, sem).wait()
                            pltpu.sync_copy(
                                buf, slab.at[idx_all.at[j]], add=True)
                            nxt = j + 2

                            @pl.when(nxt < S_NCH)
                            def _pf():
                                pltpu.async_copy(
                                    msrc.at[pl.ds(ebase + nxt * S_CH, S_CH), :],
                                    buf, sem)
                    return carry

                lax.fori_loop(0, (S_NCH + 1) // 2, pair, 0)
                plsc.subcore_barrier()
                pltpu.sync_copy(
                    slab.at[pl.ds(tid * RPT, RPT), :],
                    out_hbm.at[pl.ds(tid * RPT, RPT), k, :])
                plsc.subcore_barrier()


def kernel(features_node, features_edge, features_weights, edge_index,
           W1, b1, W2, b2, W3, b3):
    src = edge_index[0]
    tgt3 = edge_index[1].reshape(NS, S_NCH, S_CH)
    x_src = _sc_gather(features_node, src)
    m0, m1, m2, m3 = _tc_messages(features_weights, features_edge, x_src,
                                  W1, b1, W2, b2, W3, b3)
    out = _sc_scatter(m0, m1, m2, m3, tgt3)
    out0 = out[:, 0, :]
    out13 = jnp.transpose(out[:, 1:4, :], (0, 2, 1)).reshape(N_NODES, 3 * D_NODE)
    return jnp.concatenate([out0, out13], axis=1)


# speedup vs baseline: 4.7372x; 1.0052x over previous
import functools

import jax
import jax.numpy as jnp
from jax import lax
from jax.experimental import pallas as pl
from jax.experimental.pallas import tpu as pltpu
from jax.experimental.pallas import tpu_sc as plsc

N_NODES = 10000
N_EDGES = 160000
D_NODE = 128

NC = 2
NS = 16
NW = NC * NS


EPW = N_EDGES // NW
G_CH = 128
G_FULL = EPW // G_CH
G_TAIL = EPW - G_FULL * G_CH

_gather_mesh = plsc.VectorSubcoreMesh(
    core_axis_name="c", subcore_axis_name="s", num_cores=NC, num_subcores=NS)


@functools.partial(
    pl.kernel,
    out_type=jax.ShapeDtypeStruct((N_EDGES, D_NODE), jnp.float32),
    mesh=_gather_mesh,
    scratch_types=[
        pltpu.VMEM((EPW,), jnp.int32),
        pltpu.VMEM((G_CH, D_NODE), jnp.float32),
        pltpu.VMEM((G_CH, D_NODE), jnp.float32),
        pltpu.VMEM((G_TAIL, D_NODE), jnp.float32),
        pltpu.SemaphoreType.DMA,
        pltpu.SemaphoreType.DMA,
        pltpu.SemaphoreType.DMA,
    ],
)
def _sc_gather(nodes_hbm, ei3_hbm, out_hbm, idx_all, b0, b1, bt, s0, s1, st):
    c = lax.axis_index("c")
    s = lax.axis_index("s")
    wid = s * NC + c
    base = wid * EPW

    pltpu.sync_copy(ei3_hbm.at[0, wid], idx_all)
    bufs = (b0, b1)
    sems = (s0, s1)
    pltpu.async_copy(nodes_hbm.at[idx_all.at[pl.ds(0, G_CH)]], b0, s0)
    pltpu.async_copy(nodes_hbm.at[idx_all.at[pl.ds(G_CH, G_CH)]], b1, s1)
    pltpu.async_copy(
        nodes_hbm.at[idx_all.at[pl.ds(G_FULL * G_CH, G_TAIL)]], bt, st)

    def pair(t, carry):
        for b in range(2):
            j = 2 * t + b
            buf, sem = bufs[b], sems[b]

            @pl.when(j < G_FULL)
            def _chunk():
                pltpu.make_async_copy(
                    nodes_hbm.at[idx_all.at[pl.ds(0, G_CH)]], buf, sem).wait()
                pltpu.sync_copy(buf, out_hbm.at[pl.ds(base + j * G_CH, G_CH), :])
                nxt = j + 2

                @pl.when(nxt < G_FULL)
                def _pf():
                    pltpu.async_copy(
                        nodes_hbm.at[idx_all.at[pl.ds(nxt * G_CH, G_CH)]],
                        buf, sem)
        return carry

    lax.fori_loop(0, (G_FULL + 1) // 2, pair, 0)
    pltpu.make_async_copy(
        nodes_hbm.at[idx_all.at[pl.ds(0, G_TAIL)]], bt, st).wait()
    pltpu.sync_copy(bt, out_hbm.at[pl.ds(base + G_FULL * G_CH, G_TAIL), :])


BE = 2000


def _tc_body(fw_ref, fe_ref, x_ref, W1_ref, b1_ref, W2_ref, b2_ref,
             W3_ref, b3_ref, m0_ref, m1_ref, m2_ref, m3_ref):
    fw = fw_ref[...]
    h = jax.nn.silu(jnp.dot(fw, W1_ref[...], preferred_element_type=jnp.float32)
                    + b1_ref[...])
    h = jax.nn.silu(jnp.dot(h, W2_ref[...], preferred_element_type=jnp.float32)
                    + b2_ref[...])
    w = jnp.dot(h, W3_ref[...], preferred_element_type=jnp.float32) + b3_ref[...]
    x = x_ref[...]
    u0 = w[:, :D_NODE] * x
    u1 = w[:, D_NODE:] * x
    fe = fe_ref[...]
    m0_ref[...] = u0 * fe[:, 0:1]
    m1_ref[...] = u1 * fe[:, 1:2]
    m2_ref[...] = u1 * fe[:, 2:3]
    m3_ref[...] = u1 * fe[:, 3:4]


def _tc_messages(fw, fe, x_src, W1, b1, W2, b2, W3, b3):
    n_blocks = N_EDGES // BE
    full = lambda shape: pl.BlockSpec(shape, lambda i: (0, 0))
    blk = lambda cols: pl.BlockSpec((BE, cols), lambda i: (i, 0))
    out = pl.pallas_call(
        _tc_body,
        grid=(n_blocks,),
        in_specs=[
            blk(16), blk(4), blk(D_NODE),
            full((16, 64)), full((1, 64)),
            full((64, 64)), full((1, 64)),
            full((64, 256)), full((1, 256)),
        ],
        out_specs=[blk(D_NODE)] * 4,
        out_shape=[jax.ShapeDtypeStruct((N_EDGES, D_NODE), jnp.float32)] * 4,
    )(fw, fe, x_src, W1, b1.reshape(1, 64), W2, b2.reshape(1, 64),
      W3, b3.reshape(1, 256))
    return out


EPT = N_EDGES // NS
S_CH = 80
S_NCH = EPT // S_CH
RPT = N_NODES // NS
ZB = 25

_scatter_mesh = plsc.VectorSubcoreMesh(
    core_axis_name="c", subcore_axis_name="s", num_cores=NC, num_subcores=NS)


@functools.partial(
    pl.kernel,
    out_type=jax.ShapeDtypeStruct((N_NODES, 4, D_NODE), jnp.float32),
    mesh=_scatter_mesh,
    scratch_types=[
        pltpu.VMEM((S_NCH, S_CH), jnp.int32),
        pltpu.VMEM((S_CH, D_NODE), jnp.float32),
        pltpu.VMEM((S_CH, D_NODE), jnp.float32),
        pltpu.VMEM((ZB, D_NODE), jnp.float32),
        pltpu.VMEM_SHARED((N_NODES, D_NODE), jnp.float32),
        pltpu.SemaphoreType.DMA,
        pltpu.SemaphoreType.DMA,
    ],
)
def _sc_scatter(m0, m1, m2, m3, ei4_hbm, out_hbm,
                idx_all, mb0, mb1, zbuf, slab, sm0, sm1):
    c = lax.axis_index("c")
    tid = lax.axis_index("s")

    def zrow(i, carry):
        r = i // 8
        col = (i % 8) * 16
        zbuf[r, pl.ds(col, 16)] = jnp.zeros((16,), jnp.float32)
        return carry

    lax.fori_loop(0, ZB * 8, zrow, 0)

    pltpu.sync_copy(ei4_hbm.at[1, tid], idx_all)

    bufs = (mb0, mb1)
    sems = (sm0, sm1)
    planes = (m0, m1, m2, m3)
    for c_val in (0, 1):
        @pl.when(c == c_val)
        def _core():
            for kk in (0, 1):
                k = 2 * c_val + kk
                msrc = planes[k]
                ebase = tid * EPT
                for z in range(RPT // ZB):
                    pltpu.sync_copy(
                        zbuf, slab.at[pl.ds(tid * RPT + z * ZB, ZB), :])
                plsc.subcore_barrier()

                pltpu.async_copy(msrc.at[pl.ds(ebase, S_CH), :], mb0, sm0)
                pltpu.async_copy(
                    msrc.at[pl.ds(ebase + S_CH, S_CH), :], mb1, sm1)

                def pair(t, carry):
                    for b in range(2):
                        j = 2 * t + b
                        buf, sem = bufs[b], sems[b]

                        @pl.when(j < S_NCH)
                        def _chunk():
                            pltpu.make_async_copy(
                                msrc.at[pl.ds(0, S_CH), :], buf, sem).wait()
                            pltpu.sync_copy(
                                buf, slab.at[idx_all.at[j]], add=True)
                            nxt = j + 2

                            @pl.when(nxt < S_NCH)
                            def _pf():
                                pltpu.async_copy(
                                    msrc.at[pl.ds(ebase + nxt * S_CH, S_CH), :],
                                    buf, sem)
                    return carry

                lax.fori_loop(0, (S_NCH + 1) // 2, pair, 0)
                plsc.subcore_barrier()
                pltpu.sync_copy(
                    slab.at[pl.ds(tid * RPT, RPT), :],
                    out_hbm.at[pl.ds(tid * RPT, RPT), k, :])
                plsc.subcore_barrier()


def kernel(features_node, features_edge, features_weights, edge_index,
           W1, b1, W2, b2, W3, b3):
    ei3 = edge_index.reshape(2, NW, EPW)
    ei4 = edge_index.reshape(2, NS, S_NCH, S_CH)
    x_src = _sc_gather(features_node, ei3)
    m0, m1, m2, m3 = _tc_messages(features_weights, features_edge, x_src,
                                  W1, b1, W2, b2, W3, b3)
    out = _sc_scatter(m0, m1, m2, m3, ei4)
    out0 = out[:, 0, :]
    out13 = jnp.transpose(out[:, 1:4, :], (0, 2, 1)).reshape(N_NODES, 3 * D_NODE)
    return jnp.concatenate([out0, out13], axis=1)


# speedup vs baseline: 5.2962x; 1.1180x over previous
import functools

import jax
import jax.numpy as jnp
from jax import lax
from jax.experimental import pallas as pl
from jax.experimental.pallas import tpu as pltpu
from jax.experimental.pallas import tpu_sc as plsc

N_NODES = 10000
N_EDGES = 160000
D_NODE = 128

NC = 2
NS = 16
NW = NC * NS


EPW = N_EDGES // NW
G_CH = 128
G_FULL = EPW // G_CH
G_TAIL = EPW - G_FULL * G_CH

_gather_mesh = plsc.VectorSubcoreMesh(
    core_axis_name="c", subcore_axis_name="s", num_cores=NC, num_subcores=NS)


@functools.partial(
    pl.kernel,
    out_type=jax.ShapeDtypeStruct((N_EDGES, D_NODE), jnp.float32),
    mesh=_gather_mesh,
    scratch_types=[
        pltpu.VMEM((EPW,), jnp.int32),
        pltpu.VMEM((G_CH, D_NODE), jnp.float32),
        pltpu.VMEM((G_CH, D_NODE), jnp.float32),
        pltpu.VMEM((G_TAIL, D_NODE), jnp.float32),
        pltpu.SemaphoreType.DMA,
        pltpu.SemaphoreType.DMA,
        pltpu.SemaphoreType.DMA,
    ],
)
def _sc_gather(nodes_hbm, ei3_hbm, out_hbm, idx_all, b0, b1, bt, s0, s1, st):
    c = lax.axis_index("c")
    s = lax.axis_index("s")
    wid = s * NC + c
    base = wid * EPW

    pltpu.sync_copy(ei3_hbm.at[0, wid], idx_all)
    bufs = (b0, b1)
    sems = (s0, s1)
    pltpu.async_copy(nodes_hbm.at[idx_all.at[pl.ds(0, G_CH)]], b0, s0)
    pltpu.async_copy(nodes_hbm.at[idx_all.at[pl.ds(G_CH, G_CH)]], b1, s1)
    pltpu.async_copy(
        nodes_hbm.at[idx_all.at[pl.ds(G_FULL * G_CH, G_TAIL)]], bt, st)

    def pair(t, carry):
        for b in range(2):
            j = 2 * t + b
            buf, sem = bufs[b], sems[b]

            @pl.when(j < G_FULL)
            def _chunk():
                pltpu.make_async_copy(
                    nodes_hbm.at[idx_all.at[pl.ds(0, G_CH)]], buf, sem).wait()
                pltpu.sync_copy(buf, out_hbm.at[pl.ds(base + j * G_CH, G_CH), :])
                nxt = j + 2

                @pl.when(nxt < G_FULL)
                def _pf():
                    pltpu.async_copy(
                        nodes_hbm.at[idx_all.at[pl.ds(nxt * G_CH, G_CH)]],
                        buf, sem)
        return carry

    lax.fori_loop(0, (G_FULL + 1) // 2, pair, 0)
    pltpu.make_async_copy(
        nodes_hbm.at[idx_all.at[pl.ds(0, G_TAIL)]], bt, st).wait()
    pltpu.sync_copy(bt, out_hbm.at[pl.ds(base + G_FULL * G_CH, G_TAIL), :])


BE = 2000


def _tc_body(fw_ref, fe_ref, x_ref, W1_ref, b1_ref, W2_ref, b2_ref,
             W3_ref, b3_ref, m0_ref, m1_ref, m2_ref, m3_ref):
    fw = fw_ref[...]
    h = jax.nn.silu(jnp.dot(fw, W1_ref[...], preferred_element_type=jnp.float32)
                    + b1_ref[...])
    h = jax.nn.silu(jnp.dot(h, W2_ref[...], preferred_element_type=jnp.float32)
                    + b2_ref[...])
    w = jnp.dot(h, W3_ref[...], preferred_element_type=jnp.float32) + b3_ref[...]
    x = x_ref[...]
    u0 = w[:, :D_NODE] * x
    u1 = w[:, D_NODE:] * x
    fe = fe_ref[...]
    m0_ref[...] = u0 * fe[:, 0:1]
    m1_ref[...] = u1 * fe[:, 1:2]
    m2_ref[...] = u1 * fe[:, 2:3]
    m3_ref[...] = u1 * fe[:, 3:4]


def _tc_messages(fw, fe, x_src, W1, b1, W2, b2, W3, b3):
    n_blocks = N_EDGES // BE
    full = lambda shape: pl.BlockSpec(shape, lambda i: (0, 0))
    blk = lambda cols: pl.BlockSpec((BE, cols), lambda i: (i, 0))
    out = pl.pallas_call(
        _tc_body,
        grid=(n_blocks,),
        in_specs=[
            blk(16), blk(4), blk(D_NODE),
            full((16, 64)), full((1, 64)),
            full((64, 64)), full((1, 64)),
            full((64, 256)), full((1, 256)),
        ],
        out_specs=[blk(D_NODE)] * 4,
        out_shape=[jax.ShapeDtypeStruct((N_EDGES, D_NODE), jnp.float32)] * 4,
    )(fw, fe, x_src, W1, b1.reshape(1, 64), W2, b2.reshape(1, 64),
      W3, b3.reshape(1, 256))
    return out


EPT = N_EDGES // NS
S_CH = 80
S_NCH = EPT // S_CH
FIRE_K = 5
RPT = N_NODES // NS
ZB = 25

_scatter_mesh = plsc.VectorSubcoreMesh(
    core_axis_name="c", subcore_axis_name="s", num_cores=NC, num_subcores=NS)


@functools.partial(
    pl.kernel,
    out_type=jax.ShapeDtypeStruct((N_NODES, 4, D_NODE), jnp.float32),
    mesh=_scatter_mesh,
    scratch_types=[
        pltpu.VMEM((S_NCH, S_CH), jnp.int32),
        pltpu.VMEM((S_CH, D_NODE), jnp.float32),
        pltpu.VMEM((S_CH, D_NODE), jnp.float32),
        pltpu.VMEM((ZB, D_NODE), jnp.float32),
        pltpu.VMEM_SHARED((N_NODES, D_NODE), jnp.float32),
        pltpu.SemaphoreType.DMA,
        pltpu.SemaphoreType.DMA,
    ],
)
def _sc_scatter(m0, m1, m2, m3, ei4_hbm, out_hbm,
                idx_all, mb0, mb1, zbuf, slab, sm0, sm1):
    c = lax.axis_index("c")
    tid = lax.axis_index("s")

    def zrow(i, carry):
        r = i // 8
        col = (i % 8) * 16
        zbuf[r, pl.ds(col, 16)] = jnp.zeros((16,), jnp.float32)
        return carry

    lax.fori_loop(0, ZB * 8, zrow, 0)

    pltpu.sync_copy(ei4_hbm.at[1, tid], idx_all)

    bufs = (mb0, mb1)
    sems = (sm0, sm1)
    planes = (m0, m1, m2, m3)
    for c_val in (0, 1):
        @pl.when(c == c_val)
        def _core():
            for kk in (0, 1):
                k = 2 * c_val + kk
                msrc = planes[k]
                ebase = tid * EPT
                for z in range(RPT // ZB):
                    pltpu.sync_copy(
                        zbuf, slab.at[pl.ds(tid * RPT + z * ZB, ZB), :])
                plsc.subcore_barrier()

                pltpu.async_copy(msrc.at[pl.ds(ebase, S_CH), :], mb0, sm0)
                pltpu.async_copy(
                    msrc.at[pl.ds(ebase + S_CH, S_CH), :], mb1, sm1)

                def pair(t, carry):
                    for b in range(2):
                        j = 2 * t + b
                        buf, sem = bufs[b], sems[b]

                        @pl.when(j < S_NCH)
                        def _chunk():
                            pltpu.make_async_copy(
                                msrc.at[pl.ds(0, S_CH), :], buf, sem).wait()
                            pltpu.sync_copy(
                                buf, slab.at[idx_all.at[j]], add=True)
                            nxt = j + 2

                            @pl.when(nxt < S_NCH)
                            def _pf():
                                pltpu.async_copy(
                                    msrc.at[pl.ds(ebase + nxt * S_CH, S_CH), :],
                                    buf, sem)
                    return carry

                lax.fori_loop(0, (S_NCH + 1) // 2, pair, 0)
                plsc.subcore_barrier()
                pltpu.sync_copy(
                    slab.at[pl.ds(tid * RPT, RPT), :],
                    out_hbm.at[pl.ds(tid * RPT, RPT), k, :])
                plsc.subcore_barrier()


BN = 2000


def _asm_body(acc_ref, p1_ref, p2_ref, p3_ref, o_ref):
    o_ref[:, :D_NODE] = acc_ref[:, 0, :]
    o_ref[:, D_NODE:] = (
        jnp.dot(acc_ref[:, 1, :], p1_ref[...],
                preferred_element_type=jnp.float32)
        + jnp.dot(acc_ref[:, 2, :], p2_ref[...],
                  preferred_element_type=jnp.float32)
        + jnp.dot(acc_ref[:, 3, :], p3_ref[...],
                  preferred_element_type=jnp.float32))


def _tc_assemble(acc, p1, p2, p3):
    return pl.pallas_call(
        _asm_body,
        grid=(N_NODES // BN,),
        in_specs=[
            pl.BlockSpec((BN, 4, D_NODE), lambda i: (i, 0, 0)),
            pl.BlockSpec((D_NODE, 3 * D_NODE), lambda i: (0, 0)),
            pl.BlockSpec((D_NODE, 3 * D_NODE), lambda i: (0, 0)),
            pl.BlockSpec((D_NODE, 3 * D_NODE), lambda i: (0, 0)),
        ],
        out_specs=pl.BlockSpec((BN, 4 * D_NODE), lambda i: (i, 0)),
        out_shape=jax.ShapeDtypeStruct((N_NODES, 4 * D_NODE), jnp.float32),
    )(acc, p1, p2, p3)


def kernel(features_node, features_edge, features_weights, edge_index,
           W1, b1, W2, b2, W3, b3):
    ei3 = edge_index.reshape(2, NW, EPW)
    ei4 = edge_index.reshape(2, NS, S_NCH, S_CH)
    x_src = _sc_gather(features_node, ei3)
    m0, m1, m2, m3 = _tc_messages(features_weights, features_edge, x_src,
                                  W1, b1, W2, b2, W3, b3)
    acc = _sc_scatter(m0, m1, m2, m3, ei4)
    ch = jnp.arange(D_NODE)
    perms = []
    for j in range(3):
        p = jnp.zeros((D_NODE, 3 * D_NODE), jnp.float32)
        perms.append(p.at[ch, 3 * ch + j].set(1.0))
    return _tc_assemble(acc, *perms)
